# Initial kernel scaffold; baseline (speedup 1.0000x reference)
import jax
import jax.numpy as jnp
from jax.experimental import pallas as pl


def kernel(diag_seq, proc_seq, drug_seq, delta_t, service, admtype, insur, marit, seq_length, token_table, pe_dt, pe_pos):
    raise NotImplementedError("write your pallas kernel here")


# speedup vs baseline: 2.0297x; 2.0297x over previous
import functools

import jax
import jax.numpy as jnp
from jax import lax
from jax.experimental import pallas as pl
from jax.experimental.pallas import tpu as pltpu
from jax.experimental.pallas import tpu_sc as plsc

_NC = 2
_NS = 16
_NW = _NC * _NS

_B = 1024
_T = 20
_V = 35
_E = 64
_ROWS = _T * _V
_RPAD = 704
_CHUNK = 128


def _sc_gather_fn():
    spw = _B // _NW
    nfull, rem = divmod(_RPAD, _CHUNK)
    mesh = plsc.VectorSubcoreMesh(
        core_axis_name="c", subcore_axis_name="s",
        num_cores=_NC, num_subcores=_NS)

    @functools.partial(
        pl.kernel,
        out_type=jax.ShapeDtypeStruct((_B, _ROWS, _E), jnp.float32),
        mesh=mesh,
        scratch_types=[
            pltpu.VMEM((_RPAD,), jnp.int32),
            pltpu.VMEM((_RPAD, _E), jnp.float32),
            pltpu.SemaphoreType.DMA,
        ],
        compiler_params=pltpu.CompilerParams(use_tc_tiling_on_sc=False),
    )
    def sc_gather(idx_hbm, table_hbm, out_hbm, idx_v, y_v, gsem):
        wid = lax.axis_index("s") * _NC + lax.axis_index("c")

        def body(i, carry):
            bb = wid * spw + i
            pltpu.sync_copy(idx_hbm.at[bb], idx_v)
            cps = []
            for ch in range(nfull):
                cps.append(pltpu.async_copy(
                    table_hbm.at[idx_v.at[pl.ds(ch * _CHUNK, _CHUNK)]],
                    y_v.at[pl.ds(ch * _CHUNK, _CHUNK)], gsem))
            if rem:
                cps.append(pltpu.async_copy(
                    table_hbm.at[idx_v.at[pl.ds(nfull * _CHUNK, rem)]],
                    y_v.at[pl.ds(nfull * _CHUNK, rem)], gsem))
            for cp in cps:
                cp.wait()
            pltpu.sync_copy(y_v.at[pl.ds(0, _ROWS)], out_hbm.at[bb])
            return carry

        lax.fori_loop(0, spw, body, 0)

    return sc_gather


_sc_gather_cache = []


def _sc_gather(idx, table):
    if not _sc_gather_cache:
        _sc_gather_cache.append(_sc_gather_fn())
    return _sc_gather_cache[0](idx, table)


def _norm_body(g_ref, pe_ref, o_ref):
    y = g_ref[...] + pe_ref[...]
    m = jnp.mean(y, axis=1, keepdims=True)
    d = y - m
    v = jnp.mean(d * d, axis=1, keepdims=True)
    o_ref[...] = d * lax.rsqrt(v + 1e-5)


def _norm_fn():
    bb = 8
    row = _ROWS * _E
    return pl.pallas_call(
        _norm_body,
        grid=(_B // bb,),
        in_specs=[
            pl.BlockSpec((bb, row), lambda i: (i, 0)),
            pl.BlockSpec((1, row), lambda i: (0, 0)),
        ],
        out_specs=pl.BlockSpec((bb, row), lambda i: (i, 0)),
        out_shape=jax.ShapeDtypeStruct((_B, row), jnp.float32),
    )


_norm = _norm_fn()


def kernel(diag_seq, proc_seq, drug_seq, delta_t, service, admtype, insur,
           marit, seq_length, token_table, pe_dt, pe_pos):
    b, t = delta_t.shape
    vocab, e = token_table.shape

    dt = delta_t / 15.0
    len_mask = jnp.arange(t)[None, :] < seq_length[:, None]
    dt = jnp.cumsum(jnp.round(dt), axis=1) * len_mask.astype(dt.dtype)
    dt_idx = jnp.clip(dt.astype(jnp.int32), 0, pe_dt.shape[0] - 1)

    tok34 = jnp.concatenate(
        [diag_seq, proc_seq, drug_seq, service, admtype,
         insur[..., None], marit[..., None]], axis=2)
    idx35 = jnp.concatenate([tok34, (dt_idx + vocab)[..., None]], axis=2)
    idx = jnp.pad(idx35.reshape(b, _ROWS), ((0, 0), (0, _RPAD - _ROWS)))

    big_table = jnp.concatenate([token_table, pe_dt], axis=0)

    g = _sc_gather(idx, big_table)

    pe_row = jnp.broadcast_to(
        pe_pos[:t, None, :], (t, _V, e)).reshape(1, _ROWS * e)
    out = _norm(g.reshape(b, _ROWS * e), pe_row)
    return out.reshape(b, t, _V, e)


# speedup vs baseline: 2.0368x; 1.0035x over previous
import functools

import jax
import jax.numpy as jnp
from jax import lax
from jax.experimental import pallas as pl
from jax.experimental.pallas import tpu as pltpu
from jax.experimental.pallas import tpu_sc as plsc

_NC = 2
_NS = 16
_NW = _NC * _NS

_B = 1024
_T = 20
_V = 35
_E = 64
_ROWS = _T * _V
_RPAD = 704
_CHUNK = 128


def _sc_gather_fn():
    spw = _B // _NW
    nfull, rem = divmod(_RPAD, _CHUNK)
    mesh = plsc.VectorSubcoreMesh(
        core_axis_name="c", subcore_axis_name="s",
        num_cores=_NC, num_subcores=_NS)

    @functools.partial(
        pl.kernel,
        out_type=jax.ShapeDtypeStruct((_B, _ROWS, _E), jnp.float32),
        mesh=mesh,
        scratch_types=[
            pltpu.VMEM((spw, _RPAD), jnp.int32),
            pltpu.VMEM((_RPAD, _E), jnp.float32),
            pltpu.VMEM((_RPAD, _E), jnp.float32),
            pltpu.SemaphoreType.DMA,
            pltpu.SemaphoreType.DMA,
            pltpu.SemaphoreType.DMA,
            pltpu.SemaphoreType.DMA,
        ],
        compiler_params=pltpu.CompilerParams(use_tc_tiling_on_sc=False),
    )
    def sc_gather(idx_hbm, table_hbm, out_hbm, idx_v, y0, y1, g0, g1, o0, o1):
        wid = lax.axis_index("s") * _NC + lax.axis_index("c")
        base = wid * spw
        ys = (y0, y1)
        gs = (g0, g1)
        os_ = (o0, o1)

        def gather_cps(i, y_v, gsem):
            cps = []
            for ch in range(nfull):
                cps.append(pltpu.make_async_copy(
                    table_hbm.at[idx_v.at[i, pl.ds(ch * _CHUNK, _CHUNK)]],
                    y_v.at[pl.ds(ch * _CHUNK, _CHUNK)], gsem))
            if rem:
                cps.append(pltpu.make_async_copy(
                    table_hbm.at[idx_v.at[i, pl.ds(nfull * _CHUNK, rem)]],
                    y_v.at[pl.ds(nfull * _CHUNK, rem)], gsem))
            return cps

        def fire_g(i, y_v, gsem):
            for cp in gather_cps(i, y_v, gsem):
                cp.start()

        def drain_g(i, y_v, gsem):
            for cp in gather_cps(i, y_v, gsem):
                cp.wait()

        def out_cp(i, y_v, osem):
            return pltpu.make_async_copy(
                y_v.at[pl.ds(0, _ROWS)], out_hbm.at[base + i], osem)

        pltpu.sync_copy(idx_hbm.at[pl.ds(base, spw)], idx_v)
        fire_g(0, y0, g0)

        nloop = spw // 2

        def body(ii, carry):
            a = 2 * ii
            bcur = a + 1
            drain_g(a, y0, g0)
            out_cp(a, y0, o0).start()

            @pl.when(ii > 0)
            def _():
                out_cp(a - 1, y1, o1).wait()

            fire_g(bcur, y1, g1)
            drain_g(bcur, y1, g1)
            out_cp(bcur, y1, o1).start()
            out_cp(a, y0, o0).wait()

            @pl.when(ii + 1 < nloop)
            def _():
                fire_g(a + 2, y0, g0)

            return carry

        lax.fori_loop(0, nloop, body, 0)
        out_cp(spw - 1, y1, o1).wait()

    return sc_gather


_sc_gather_cache = []


def _sc_gather(idx, table):
    if not _sc_gather_cache:
        _sc_gather_cache.append(_sc_gather_fn())
    return _sc_gather_cache[0](idx, table)


def _norm_body(g_ref, pe_ref, o_ref):
    y = g_ref[...] + pe_ref[...]
    m = jnp.mean(y, axis=1, keepdims=True)
    d = y - m
    v = jnp.mean(d * d, axis=1, keepdims=True)
    o_ref[...] = d * lax.rsqrt(v + 1e-5)


def _norm_fn():
    bb = 8
    row = _ROWS * _E
    return pl.pallas_call(
        _norm_body,
        grid=(_B // bb,),
        in_specs=[
            pl.BlockSpec((bb, row), lambda i: (i, 0)),
            pl.BlockSpec((1, row), lambda i: (0, 0)),
        ],
        out_specs=pl.BlockSpec((bb, row), lambda i: (i, 0)),
        out_shape=jax.ShapeDtypeStruct((_B, row), jnp.float32),
    )


_norm = _norm_fn()


def kernel(diag_seq, proc_seq, drug_seq, delta_t, service, admtype, insur,
           marit, seq_length, token_table, pe_dt, pe_pos):
    b, t = delta_t.shape
    vocab, e = token_table.shape

    dt = delta_t / 15.0
    len_mask = jnp.arange(t)[None, :] < seq_length[:, None]
    dt = jnp.cumsum(jnp.round(dt), axis=1) * len_mask.astype(dt.dtype)
    dt_idx = jnp.clip(dt.astype(jnp.int32), 0, pe_dt.shape[0] - 1)

    tok34 = jnp.concatenate(
        [diag_seq, proc_seq, drug_seq, service, admtype,
         insur[..., None], marit[..., None]], axis=2)
    idx35 = jnp.concatenate([tok34, (dt_idx + vocab)[..., None]], axis=2)
    idx = jnp.pad(idx35.reshape(b, _ROWS), ((0, 0), (0, _RPAD - _ROWS)))

    big_table = jnp.concatenate([token_table, pe_dt], axis=0)

    g = _sc_gather(idx, big_table)

    pe_row = jnp.broadcast_to(
        pe_pos[:t, None, :], (t, _V, e)).reshape(1, _ROWS * e)
    out = _norm(g.reshape(b, _ROWS * e), pe_row)
    return out.reshape(b, t, _V, e)
